# Initial kernel scaffold; baseline (speedup 1.0000x reference)
import jax
import jax.numpy as jnp
from jax.experimental import pallas as pl


def kernel(style_ids, frame_ids, type, latents, style_latents_mu):
    raise NotImplementedError("write your pallas kernel here")


# speedup vs baseline: 1.1101x; 1.1101x over previous
import functools

import jax
import jax.numpy as jnp
from jax import lax
from jax.experimental import pallas as pl
from jax.experimental.pallas import tpu as pltpu
from jax.experimental.pallas import tpu_sc as plsc

_STYLE_NUM = 100
_FRAME_NUM = 1000
_LATENT_DIM = 64
_B = 16384

_NC = 2
_NS = 16
_NW = _NC * _NS
_BPW = _B // _NW
_L = 16
_CH = 128
_NCHUNK = _BPW // _CH


def _gather_body(style_hbm, frame_hbm, lat_hbm, out_hbm,
                 sid_v, fid_v, idx_v, rows_v, sem):
    wid = lax.axis_index("s") * _NC + lax.axis_index("c")
    base = wid * _BPW

    pltpu.sync_copy(style_hbm.at[pl.ds(base, _BPW)], sid_v)
    pltpu.sync_copy(frame_hbm.at[pl.ds(base, _BPW)], fid_v)

    for j in range(_NCHUNK):
        for i in range(_CH // _L):
            off = j * _CH + i * _L
            s = sid_v[pl.ds(off, _L)]
            f = fid_v[pl.ds(off, _L)]
            idx_v[j, pl.ds(i * _L, _L)] = s * _FRAME_NUM + f

    copies = [
        pltpu.async_copy(lat_hbm.at[idx_v.at[j]],
                         rows_v.at[pl.ds(j * _CH, _CH)], sem)
        for j in range(_NCHUNK)
    ]
    for c in copies:
        c.wait()

    pltpu.sync_copy(rows_v, out_hbm.at[pl.ds(base, _BPW)])


@jax.jit
def _sc_gather(style_ids, frame_ids, lat_flat):
    mesh = plsc.VectorSubcoreMesh(core_axis_name="c", subcore_axis_name="s")
    return pl.kernel(
        _gather_body,
        out_type=jax.ShapeDtypeStruct((_B, _LATENT_DIM), jnp.float32),
        mesh=mesh,
        scratch_types=[
            pltpu.VMEM((_BPW,), jnp.int32),
            pltpu.VMEM((_BPW,), jnp.int32),
            pltpu.VMEM((_NCHUNK, _CH), jnp.int32),
            pltpu.VMEM((_BPW, _LATENT_DIM), jnp.float32),
            pltpu.SemaphoreType.DMA,
        ],
        compiler_params=pltpu.CompilerParams(use_tc_tiling_on_sc=False),
    )(style_ids, frame_ids, lat_flat)


def kernel(style_ids, frame_ids, type, latents, style_latents_mu):
    del type, style_latents_mu
    lat_flat = latents.reshape(-1, _LATENT_DIM)
    return _sc_gather(style_ids, frame_ids, lat_flat)


# speedup vs baseline: 1.5252x; 1.3739x over previous
import functools

import jax
import jax.numpy as jnp
from jax import lax
from jax.experimental import pallas as pl
from jax.experimental.pallas import tpu as pltpu
from jax.experimental.pallas import tpu_sc as plsc

_STYLE_NUM = 100
_FRAME_NUM = 1000
_LATENT_DIM = 64
_B = 16384

_NC = 2
_NS = 16
_NW = _NC * _NS
_BPW = _B // _NW
_CH = 64


def _gather_body(style_hbm, frame_hbm, lat_hbm, out_hbm,
                 sid_v, fid_v, rows_v, sem):
    wid = lax.axis_index("s") * _NC + lax.axis_index("c")
    base = wid * _BPW

    pltpu.sync_copy(style_hbm.at[pl.ds(base, _BPW)], sid_v)
    pltpu.sync_copy(frame_hbm.at[pl.ds(base, _BPW)], fid_v)

    def chunk(j, carry):
        def fire_group(g, c):
            k0 = j * _CH + g * 16
            s16 = sid_v[pl.ds(k0, 16)]
            f16 = fid_v[pl.ds(k0, 16)]
            for i in range(16):
                pltpu.async_copy(lat_hbm.at[s16[i], f16[i]],
                                 rows_v.at[k0 + i], sem)
            return c

        lax.fori_loop(0, _CH // 16, fire_group, 0)

        def drain(i, c):
            k = j * _CH + i
            pltpu.make_async_copy(lat_hbm.at[0, 0], rows_v.at[k], sem).wait()
            return c

        lax.fori_loop(0, _CH, drain, 0)
        return carry

    lax.fori_loop(0, _BPW // _CH, chunk, 0)

    pltpu.sync_copy(rows_v, out_hbm.at[pl.ds(base, _BPW)])


@jax.jit
def _sc_gather(style_ids, frame_ids, latents):
    mesh = plsc.VectorSubcoreMesh(core_axis_name="c", subcore_axis_name="s")
    return pl.kernel(
        _gather_body,
        out_type=jax.ShapeDtypeStruct((_B, _LATENT_DIM), jnp.float32),
        mesh=mesh,
        scratch_types=[
            pltpu.VMEM((_BPW,), jnp.int32),
            pltpu.VMEM((_BPW,), jnp.int32),
            pltpu.VMEM((_BPW, _LATENT_DIM), jnp.float32),
            pltpu.SemaphoreType.DMA,
        ],
        compiler_params=pltpu.CompilerParams(use_tc_tiling_on_sc=True),
    )(style_ids, frame_ids, latents)


def kernel(style_ids, frame_ids, type, latents, style_latents_mu):
    del type, style_latents_mu
    return _sc_gather(style_ids, frame_ids, latents)


# speedup vs baseline: 1.7832x; 1.1692x over previous
import functools

import jax
import jax.numpy as jnp
from jax import lax
from jax.experimental import pallas as pl
from jax.experimental.pallas import tpu as pltpu
from jax.experimental.pallas import tpu_sc as plsc

_STYLE_NUM = 100
_FRAME_NUM = 1000
_LATENT_DIM = 64
_B = 16384

_NC = 2
_NS = 16
_NW = _NC * _NS
_BPW = _B // _NW
_CH = 64


def _gather_body(style_hbm, frame_hbm, lat_hbm, out_hbm,
                 sid_v, fid_v, flat_v, rows_v, sem):
    wid = lax.axis_index("s") * _NC + lax.axis_index("c")
    base = wid * _BPW

    pltpu.sync_copy(style_hbm.at[pl.ds(base, _BPW)], sid_v)
    pltpu.sync_copy(frame_hbm.at[pl.ds(base, _BPW)], fid_v)

    def ids(g, c):
        s = sid_v[pl.ds(g * 16, 16)]
        f = fid_v[pl.ds(g * 16, 16)]
        flat_v[pl.ds(g * 16, 16)] = s * _FRAME_NUM + f
        return c

    lax.fori_loop(0, _BPW // 16, ids, 0)

    def chunk(j, carry):
        def fire_group(g, c):
            k0 = j * _CH + g * 16
            r16 = flat_v[pl.ds(k0, 16)]
            for i in range(16):
                pltpu.async_copy(lat_hbm.at[r16[i]], rows_v.at[k0 + i], sem)
            return c

        lax.fori_loop(0, _CH // 16, fire_group, 0)

        pltpu.make_async_copy(
            lat_hbm.at[pl.ds(0, _CH)],
            rows_v.at[pl.ds(j * _CH, _CH)], sem).wait()
        return carry

    lax.fori_loop(0, _BPW // _CH, chunk, 0)

    pltpu.sync_copy(rows_v, out_hbm.at[pl.ds(base, _BPW)])


@jax.jit
def _sc_gather(style_ids, frame_ids, lat_flat):
    mesh = plsc.VectorSubcoreMesh(core_axis_name="c", subcore_axis_name="s")
    return pl.kernel(
        _gather_body,
        out_type=jax.ShapeDtypeStruct((_B, _LATENT_DIM), jnp.float32),
        mesh=mesh,
        scratch_types=[
            pltpu.VMEM((_BPW,), jnp.int32),
            pltpu.VMEM((_BPW,), jnp.int32),
            pltpu.VMEM((_BPW,), jnp.int32),
            pltpu.VMEM((_BPW, _LATENT_DIM), jnp.float32),
            pltpu.SemaphoreType.DMA,
        ],
        compiler_params=pltpu.CompilerParams(use_tc_tiling_on_sc=True),
    )(style_ids, frame_ids, lat_flat)


def kernel(style_ids, frame_ids, type, latents, style_latents_mu):
    del type, style_latents_mu
    return _sc_gather(style_ids, frame_ids, latents.reshape(-1, _LATENT_DIM))


# speedup vs baseline: 1.9172x; 1.0751x over previous
import functools

import jax
import jax.numpy as jnp
from jax import lax
from jax.experimental import pallas as pl
from jax.experimental.pallas import tpu as pltpu
from jax.experimental.pallas import tpu_sc as plsc

_STYLE_NUM = 100
_FRAME_NUM = 1000
_LATENT_DIM = 64
_B = 16384

_NC = 2
_NS = 16
_NW = _NC * _NS
_BPW = _B // _NW
_CH = 64


def _gather_body(style_hbm, frame_hbm, lat_hbm, out_hbm,
                 sid_v, fid_v, flat_v, rows_v, sem, osem):
    wid = lax.axis_index("s") * _NC + lax.axis_index("c")
    base = wid * _BPW

    pltpu.sync_copy(style_hbm.at[pl.ds(base, _BPW)], sid_v)
    pltpu.sync_copy(frame_hbm.at[pl.ds(base, _BPW)], fid_v)

    def ids(g, c):
        s = sid_v[pl.ds(g * 16, 16)]
        f = fid_v[pl.ds(g * 16, 16)]
        flat_v[pl.ds(g * 16, 16)] = s * _FRAME_NUM + f
        return c

    lax.fori_loop(0, _BPW // 16, ids, 0)

    def fire_chunk(j):
        def fire_group(g, c):
            k0 = j * _CH + g * 16
            r16 = flat_v[pl.ds(k0, 16)]
            for i in range(16):
                pltpu.async_copy(lat_hbm.at[r16[i]], rows_v.at[k0 + i], sem)
            return c

        lax.fori_loop(0, _CH // 16, fire_group, 0)

    def drain_chunk(j):
        pltpu.make_async_copy(
            lat_hbm.at[pl.ds(0, _CH)],
            rows_v.at[pl.ds(j * _CH, _CH)], sem).wait()
        pltpu.async_copy(rows_v.at[pl.ds(j * _CH, _CH)],
                         out_hbm.at[pl.ds(base + j * _CH, _CH)], osem)

    fire_chunk(0)

    def step(j, carry):
        fire_chunk(j + 1)
        drain_chunk(j)
        return carry

    lax.fori_loop(0, _BPW // _CH - 1, step, 0)
    drain_chunk(_BPW // _CH - 1)

    pltpu.make_async_copy(rows_v, out_hbm.at[pl.ds(base, _BPW)], osem).wait()


@jax.jit
def _sc_gather(style_ids, frame_ids, lat_flat):
    mesh = plsc.VectorSubcoreMesh(core_axis_name="c", subcore_axis_name="s")
    return pl.kernel(
        _gather_body,
        out_type=jax.ShapeDtypeStruct((_B, _LATENT_DIM), jnp.float32),
        mesh=mesh,
        scratch_types=[
            pltpu.VMEM((_BPW,), jnp.int32),
            pltpu.VMEM((_BPW,), jnp.int32),
            pltpu.VMEM((_BPW,), jnp.int32),
            pltpu.VMEM((_BPW, _LATENT_DIM), jnp.float32),
            pltpu.SemaphoreType.DMA,
            pltpu.SemaphoreType.DMA,
        ],
        compiler_params=pltpu.CompilerParams(use_tc_tiling_on_sc=True),
    )(style_ids, frame_ids, lat_flat)


def kernel(style_ids, frame_ids, type, latents, style_latents_mu):
    del type, style_latents_mu
    return _sc_gather(style_ids, frame_ids, latents.reshape(-1, _LATENT_DIM))
